# baseline (speedup 1.0000x reference)
import functools

import jax
import jax.numpy as jnp
from jax import lax
from jax.experimental import pallas as pl
from jax.experimental.pallas import tpu as pltpu
from jax.experimental.pallas import tpu_sc as plsc

B = 16384
F = 26
NC = 1
NS = 16
NW = NC * NS
BPW = B // NW
IPW = BPW * F
CHUNKS = BPW // 16


NQ = 4
QI = IPW // NQ


def _lr_kernel(x_hbm, t_hbm, bias_hbm, out_hbm, idx_v, vals_v, acc_v, bias_v,
               sem):
    wid = lax.axis_index("s") * NC + lax.axis_index("c")
    base = wid * IPW

    pltpu.sync_copy(x_hbm.at[pl.ds(base, IPW)], idx_v)
    pltpu.sync_copy(bias_hbm, bias_v)
    copies = [
        pltpu.async_copy(
            t_hbm.at[idx_v.at[pl.ds(q * QI, QI)]],
            vals_v.at[pl.ds(q * QI, QI)],
            sem,
        )
        for q in range(NQ)
    ]
    for c in copies:
        c.wait()

    lane = lax.iota(jnp.int32, 16)
    bias_vec = bias_v[...]

    def chunk_body(c, _):
        rowbase = c * (16 * F)
        idx0 = rowbase + lane * F
        acc = bias_vec
        for f in range(F):
            acc = acc + plsc.load_gather(vals_v, [idx0 + f])
        acc_v[pl.ds(c * 16, 16)] = acc
        return 0

    lax.fori_loop(0, CHUNKS, chunk_body, 0)
    pltpu.sync_copy(acc_v, out_hbm.at[pl.ds(wid * BPW, BPW)])


@jax.jit
def _lr(x_flat, t_flat, bias16):
    mesh = plsc.VectorSubcoreMesh(core_axis_name="c", subcore_axis_name="s",
                                  num_cores=NC)
    f = functools.partial(
        pl.kernel,
        out_type=jax.ShapeDtypeStruct((B,), jnp.float32),
        mesh=mesh,
        scratch_types=[
            pltpu.VMEM((IPW,), jnp.int32),
            pltpu.VMEM((IPW,), jnp.float32),
            pltpu.VMEM((BPW,), jnp.float32),
            pltpu.VMEM((16,), jnp.float32),
            pltpu.SemaphoreType.DMA,
        ],
        compiler_params=pltpu.CompilerParams(needs_layout_passes=False),
    )(_lr_kernel)
    return f(x_flat, t_flat, bias16)


def kernel(X, table, bias):
    x_flat = X.reshape(-1)
    t_flat = table.reshape(-1)
    bias16 = jnp.broadcast_to(bias, (16,))
    out = _lr(x_flat, t_flat, bias16)
    return out.reshape(B, 1)


# speedup vs baseline: 1.1500x; 1.1500x over previous
import functools

import jax
import jax.numpy as jnp
from jax import lax
from jax.experimental import pallas as pl
from jax.experimental.pallas import tpu as pltpu
from jax.experimental.pallas import tpu_sc as plsc

B = 16384
F = 26
NC = 2
NS = 16
NW = NC * NS
BPW = B // NW
IPW = BPW * F
NQ = 4
QI = IPW // NQ
QROWS = BPW // NQ
QCHUNKS = QROWS // 16


def _lr_kernel(x_hbm, t_hbm, bias_hbm, out_hbm, idx_v, vals_v, acc_v, bias_v,
               sem_i, sem_g):
    wid = lax.axis_index("s") * NC + lax.axis_index("c")
    base = wid * IPW

    pltpu.sync_copy(bias_hbm, bias_v)
    idx_copies = [
        pltpu.async_copy(
            x_hbm.at[pl.ds(base + q * QI, QI)],
            idx_v.at[pl.ds(q * QI, QI)],
            sem_i[q],
        )
        for q in range(NQ)
    ]
    gathers = []
    for q in range(NQ):
        idx_copies[q].wait()
        gathers.append(
            pltpu.async_copy(
                t_hbm.at[idx_v.at[pl.ds(q * QI, QI)]],
                vals_v.at[pl.ds(q * QI, QI)],
                sem_g[q],
            ))

    lane = lax.iota(jnp.int32, 16)
    bias_vec = bias_v[...]
    lane_f = lane * F

    for q in range(NQ):
        gathers[q].wait()

        def chunk_body(c, _, q=q):
            rowbase = (q * QROWS + c * 16) * F
            idx0 = rowbase + lane_f
            acc = bias_vec
            for f in range(F):
                acc = acc + plsc.load_gather(vals_v, [idx0 + f])
            acc_v[pl.ds(q * QROWS + c * 16, 16)] = acc
            return 0

        lax.fori_loop(0, QCHUNKS, chunk_body, 0)

    pltpu.sync_copy(acc_v, out_hbm.at[pl.ds(wid * BPW, BPW)])


@jax.jit
def _lr(x_flat, t_flat, bias16):
    mesh = plsc.VectorSubcoreMesh(core_axis_name="c", subcore_axis_name="s",
                                  num_cores=NC)
    f = functools.partial(
        pl.kernel,
        out_type=jax.ShapeDtypeStruct((B,), jnp.float32),
        mesh=mesh,
        scratch_types=[
            pltpu.VMEM((IPW,), jnp.int32),
            pltpu.VMEM((IPW,), jnp.float32),
            pltpu.VMEM((BPW,), jnp.float32),
            pltpu.VMEM((16,), jnp.float32),
            [pltpu.SemaphoreType.DMA] * NQ,
            [pltpu.SemaphoreType.DMA] * NQ,
        ],
        compiler_params=pltpu.CompilerParams(needs_layout_passes=False),
    )(_lr_kernel)
    return f(x_flat, t_flat, bias16)


def kernel(X, table, bias):
    x_flat = X.reshape(-1)
    t_flat = table.reshape(-1)
    bias16 = jnp.broadcast_to(bias, (16,))
    out = _lr(x_flat, t_flat, bias16)
    return out.reshape(B, 1)


# speedup vs baseline: 1.8410x; 1.6009x over previous
import functools

import jax
import jax.numpy as jnp
from jax import lax
from jax.experimental import pallas as pl
from jax.experimental.pallas import tpu as pltpu
from jax.experimental.pallas import tpu_sc as plsc

B = 16384
F = 26
NC = 2
NS = 16
NW = NC * NS
BPW = B // NW
IPW = BPW * F
NQ = 4
QI = IPW // NQ
QROWS = BPW // NQ
QCHUNKS = QROWS // 16
VOCAB_PAD = 1000448


def _lr_kernel(x_hbm, t_hbm, bias_hbm, out_hbm, idx_v, vals_v, acc_v, bias_v,
               sem_i, sem_g):
    wid = lax.axis_index("s") * NC + lax.axis_index("c")
    base = wid * IPW

    pltpu.sync_copy(bias_hbm, bias_v)
    idx_copies = [
        pltpu.async_copy(
            x_hbm.at[pl.ds(base + q * QI, QI)],
            idx_v.at[pl.ds(q * QI, QI)],
            sem_i[q],
        )
        for q in range(NQ)
    ]
    gathers = []
    for q in range(NQ):
        idx_copies[q].wait()
        gathers.append(
            pltpu.async_copy(
                t_hbm.at[idx_v.at[pl.ds(q * QI, QI)]],
                vals_v.at[pl.ds(q * QI, QI)],
                sem_g[q],
            ))

    lane = lax.iota(jnp.int32, 16)
    bias_vec = bias_v[...]
    lane_f = lane * F

    for q in range(NQ):
        gathers[q].wait()

        def chunk_body(c, _, q=q):
            rowbase = (q * QROWS + c * 16) * F
            idx0 = rowbase + lane_f
            acc = bias_vec
            for f in range(F):
                acc = acc + plsc.load_gather(vals_v, [idx0 + f])
            acc_v[pl.ds(q * QROWS + c * 16, 16)] = acc
            return 0

        lax.fori_loop(0, QCHUNKS, chunk_body, 0)

    pltpu.sync_copy(acc_v, out_hbm.at[pl.ds(wid * BPW, BPW)])


@jax.jit
def _lr(x_flat, t_flat, bias16):
    mesh = plsc.VectorSubcoreMesh(core_axis_name="c", subcore_axis_name="s",
                                  num_cores=NC)
    f = functools.partial(
        pl.kernel,
        out_type=jax.ShapeDtypeStruct((B,), jnp.float32),
        mesh=mesh,
        scratch_types=[
            pltpu.VMEM((IPW,), jnp.int32),
            pltpu.VMEM((IPW,), jnp.float32),
            pltpu.VMEM((BPW,), jnp.float32),
            pltpu.VMEM((16,), jnp.float32),
            [pltpu.SemaphoreType.DMA] * NQ,
            [pltpu.SemaphoreType.DMA] * NQ,
        ],
        compiler_params=pltpu.CompilerParams(needs_layout_passes=False),
    )(_lr_kernel)
    return f(x_flat, t_flat, bias16)


def kernel(X, table, bias):
    x_flat = X.reshape(-1)
    t_flat = jnp.pad(table, ((0, VOCAB_PAD - 1000000), (0, 0))).reshape(-1)
    bias16 = jnp.broadcast_to(bias, (16,))
    out = _lr(x_flat, t_flat, bias16)
    return out.reshape(B, 1)


# speedup vs baseline: 2.1745x; 1.1811x over previous
import functools

import jax
import jax.numpy as jnp
from jax import lax
from jax.experimental import pallas as pl
from jax.experimental.pallas import tpu as pltpu
from jax.experimental.pallas import tpu_sc as plsc

B = 16384
F = 26
NC = 2
NS = 16
NW = NC * NS
BPW = B // NW
CHUNKS = BPW // 16
VOCAB_PAD = 1000448


def _lr_kernel(x_hbm, t_hbm, bias_hbm, out_hbm, idx_v, vals_v, acc_v, bias_v,
               sem_i, sem_g):
    wid = lax.axis_index("s") * NC + lax.axis_index("c")
    base = wid * BPW

    pltpu.sync_copy(bias_hbm, bias_v)
    idx_copies = [
        pltpu.async_copy(
            x_hbm.at[f, pl.ds(base, BPW)],
            idx_v.at[pl.ds(f * BPW, BPW)],
            sem_i,
        )
        for f in range(F)
    ]
    for c in idx_copies:
        c.wait()
    gathers = [
        pltpu.async_copy(
            t_hbm.at[idx_v.at[pl.ds(f * BPW, BPW)]],
            vals_v.at[pl.ds(f * BPW, BPW)],
            sem_g,
        )
        for f in range(F)
    ]

    bias_vec = bias_v[...]

    def init_body(c, _):
        acc_v[pl.ds(c * 16, 16)] = bias_vec
        return 0

    lax.fori_loop(0, CHUNKS, init_body, 0)

    for g in gathers:
        g.wait()

    for f in range(F):

        def acc_body(c, _, f=f):
            sl = pl.ds(c * 16, 16)
            acc_v[sl] = acc_v[sl] + vals_v[pl.ds(f * BPW + c * 16, 16)]
            return 0

        lax.fori_loop(0, CHUNKS, acc_body, 0)

    pltpu.sync_copy(acc_v, out_hbm.at[pl.ds(base, BPW)])


@jax.jit
def _lr(x_t, t_flat, bias16):
    mesh = plsc.VectorSubcoreMesh(core_axis_name="c", subcore_axis_name="s",
                                  num_cores=NC)
    f = functools.partial(
        pl.kernel,
        out_type=jax.ShapeDtypeStruct((B,), jnp.float32),
        mesh=mesh,
        scratch_types=[
            pltpu.VMEM((F * BPW,), jnp.int32),
            pltpu.VMEM((F * BPW,), jnp.float32),
            pltpu.VMEM((BPW,), jnp.float32),
            pltpu.VMEM((16,), jnp.float32),
            pltpu.SemaphoreType.DMA,
            pltpu.SemaphoreType.DMA,
        ],
        compiler_params=pltpu.CompilerParams(needs_layout_passes=False),
    )(_lr_kernel)
    return f(x_t, t_flat, bias16)


def kernel(X, table, bias):
    x_t = X.T
    t_flat = jnp.pad(table, ((0, VOCAB_PAD - 1000000), (0, 0))).reshape(-1)
    bias16 = jnp.broadcast_to(bias, (16,))
    out = _lr(x_t, t_flat, bias16)
    return out.reshape(B, 1)


# speedup vs baseline: 2.3544x; 1.0828x over previous
import functools

import jax
import jax.numpy as jnp
from jax import lax
from jax.experimental import pallas as pl
from jax.experimental.pallas import tpu as pltpu
from jax.experimental.pallas import tpu_sc as plsc

B = 16384
F = 26
NC = 2
NS = 16
NW = NC * NS
BPW = B // NW
CHUNKS = BPW // 16
VOCAB_PAD = 1000448


def _lr_kernel(x_hbm, t_hbm, bias_hbm, out_hbm, idx_v, vals_v, acc_v, bias_v,
               sem_i, sem_g):
    wid = lax.axis_index("s") * NC + lax.axis_index("c")
    base = wid * BPW

    pltpu.sync_copy(bias_hbm, bias_v)
    idx_copies = [
        pltpu.async_copy(
            x_hbm.at[f, pl.ds(base, BPW)],
            idx_v.at[pl.ds(f * BPW, BPW)],
            sem_i,
        )
        for f in range(F)
    ]
    for c in idx_copies:
        c.wait()
    gathers = [
        pltpu.async_copy(
            t_hbm.at[idx_v.at[pl.ds(f * BPW, BPW)]],
            vals_v.at[pl.ds(f * BPW, BPW)],
            sem_g,
        )
        for f in range(F)
    ]

    bias_vec = bias_v[...]

    for g in gathers:
        g.wait()

    def chunk_body(c, _):
        col = c * 16
        acc = bias_vec
        for f in range(F):
            acc = acc + vals_v[pl.ds(f * BPW + col, 16)]
        acc_v[pl.ds(col, 16)] = acc
        return 0

    lax.fori_loop(0, CHUNKS, chunk_body, 0)

    pltpu.sync_copy(acc_v, out_hbm.at[pl.ds(base, BPW)])


@jax.jit
def _lr(x_t, t_flat, bias16):
    mesh = plsc.VectorSubcoreMesh(core_axis_name="c", subcore_axis_name="s",
                                  num_cores=NC)
    f = functools.partial(
        pl.kernel,
        out_type=jax.ShapeDtypeStruct((B,), jnp.float32),
        mesh=mesh,
        scratch_types=[
            pltpu.VMEM((F * BPW,), jnp.int32),
            pltpu.VMEM((F * BPW,), jnp.float32),
            pltpu.VMEM((BPW,), jnp.float32),
            pltpu.VMEM((16,), jnp.float32),
            pltpu.SemaphoreType.DMA,
            pltpu.SemaphoreType.DMA,
        ],
        compiler_params=pltpu.CompilerParams(needs_layout_passes=False),
    )(_lr_kernel)
    return f(x_t, t_flat, bias16)


def kernel(X, table, bias):
    x_t = X.T
    t_flat = jnp.pad(table, ((0, VOCAB_PAD - 1000000), (0, 0))).reshape(-1)
    bias16 = jnp.broadcast_to(bias, (16,))
    out = _lr(x_t, t_flat, bias16)
    return out.reshape(B, 1)


# speedup vs baseline: 2.3646x; 1.0043x over previous
import functools

import jax
import jax.numpy as jnp
from jax import lax
from jax.experimental import pallas as pl
from jax.experimental.pallas import tpu as pltpu
from jax.experimental.pallas import tpu_sc as plsc

B = 16384
F = 26
NC = 2
NS = 16
NW = NC * NS
BPW = B // NW
CHUNKS = BPW // 16
VOCAB_PAD = 1000448
GROUPS = [range(0, 7), range(7, 14), range(14, 20), range(20, 26)]


def _lr_kernel(x_hbm, t_hbm, bias_hbm, out_hbm, idx_v, vals_v, acc_v,
               bias_v, sem_i, sem_g):
    wid = lax.axis_index("s") * NC + lax.axis_index("c")
    base = wid * BPW

    pltpu.sync_copy(bias_hbm, bias_v)
    idx_copies = [
        pltpu.async_copy(
            x_hbm.at[f, pl.ds(base, BPW)],
            idx_v.at[pl.ds(f * BPW, BPW)],
            sem_i[g],
        )
        for g, fs in enumerate(GROUPS)
        for f in fs
    ]
    gathers = []
    k = 0
    for g, fs in enumerate(GROUPS):
        for _ in fs:
            idx_copies[k].wait()
            k += 1
        for f in fs:
            gathers.append(
                pltpu.async_copy(
                    t_hbm.at[idx_v.at[pl.ds(f * BPW, BPW)]],
                    vals_v.at[pl.ds(f * BPW, BPW)],
                    sem_g[g],
                ))

    bias_vec = bias_v[...]

    k = 0
    for g, fs in enumerate(GROUPS):
        for _ in fs:
            gathers[k].wait()
            k += 1

        def group_body(c, _, g=g, fs=fs):
            col = c * 16
            sl = pl.ds(col, 16)
            acc = bias_vec if g == 0 else acc_v[sl]
            for f in fs:
                acc = acc + vals_v[pl.ds(f * BPW + col, 16)]
            acc_v[sl] = acc
            return 0

        lax.fori_loop(0, CHUNKS, group_body, 0)

    pltpu.sync_copy(acc_v, out_hbm.at[pl.ds(base, BPW)])


@jax.jit
def _lr(x_t, t_flat, bias16):
    mesh = plsc.VectorSubcoreMesh(core_axis_name="c", subcore_axis_name="s",
                                  num_cores=NC)
    f = functools.partial(
        pl.kernel,
        out_type=jax.ShapeDtypeStruct((B,), jnp.float32),
        mesh=mesh,
        scratch_types=[
            pltpu.VMEM((F * BPW,), jnp.int32),
            pltpu.VMEM((F * BPW,), jnp.float32),
            pltpu.VMEM((BPW,), jnp.float32),
            pltpu.VMEM((16,), jnp.float32),
            [pltpu.SemaphoreType.DMA] * len(GROUPS),
            [pltpu.SemaphoreType.DMA] * len(GROUPS),
        ],
        compiler_params=pltpu.CompilerParams(needs_layout_passes=False),
    )(_lr_kernel)
    return f(x_t, t_flat, bias16)


def kernel(X, table, bias):
    x_t = X.T
    t_flat = jnp.pad(table, ((0, VOCAB_PAD - 1000000), (0, 0))).reshape(-1)
    bias16 = jnp.broadcast_to(bias, (16,))
    out = _lr(x_t, t_flat, bias16)
    return out.reshape(B, 1)
